# Initial kernel scaffold; baseline (speedup 1.0000x reference)
import jax
import jax.numpy as jnp
from jax.experimental import pallas as pl


def kernel(x, edge_index, W1, a_src1, a_dst1, b1, W2, a_src2, a_dst2, b2):
    raise NotImplementedError("write your pallas kernel here")


# speedup vs baseline: 17.3289x; 17.3289x over previous
import functools

import jax
import jax.numpy as jnp
from jax import lax
from jax.experimental import pallas as pl
from jax.experimental.pallas import tpu as pltpu
from jax.experimental.pallas import tpu_sc as plsc

N_NODES = 10000
N_EDGES = 320000
D = 128

NPAD = 10240
E_TOT = N_EDGES + N_NODES
NW = 32
CHUNK = 128
CPW = -(-E_TOT // (NW * CHUNK))
EPAD = NW * CHUNK * CPW
ROWS_PER_TILE = NPAD // 16


def _tc_project(x_blk, w_blk, a_blk, h_out, av_out):
    h = jnp.dot(x_blk[...], w_blk[...], preferred_element_type=jnp.float32, precision=jax.lax.Precision.HIGHEST)
    h_out[...] = h
    av_out[...] = jnp.dot(h, a_blk[...], preferred_element_type=jnp.float32, precision=jax.lax.Precision.HIGHEST)


def _project(xp, W, A):
    grid = NPAD // 512
    return pl.pallas_call(
        _tc_project,
        grid=(grid,),
        in_specs=[
            pl.BlockSpec((512, D), lambda i: (i, 0)),
            pl.BlockSpec((D, D), lambda i: (0, 0)),
            pl.BlockSpec((D, 2), lambda i: (0, 0)),
        ],
        out_specs=[
            pl.BlockSpec((512, D), lambda i: (i, 0)),
            pl.BlockSpec((512, 2), lambda i: (i, 0)),
        ],
        out_shape=[
            jax.ShapeDtypeStruct((NPAD, D), jnp.float32),
            jax.ShapeDtypeStruct((NPAD, 2), jnp.float32),
        ],
    )(xp, W, A)


def _recip(den):
    r = 1.0 / den
    return r * (2.0 - den * r)


def _tc_mid(a0, a1, d0, d1, bb, w_blk, a_blk, h_out, av_out):
    den = d0[...] + d1[...] + 1e-16
    hin = (a0[...] + a1[...]) * _recip(den) + bb[...]
    hin = jnp.maximum(hin, 0.01 * hin)
    h = jnp.dot(hin, w_blk[...], preferred_element_type=jnp.float32, precision=jax.lax.Precision.HIGHEST)
    h_out[...] = h
    av_out[...] = jnp.dot(h, a_blk[...], preferred_element_type=jnp.float32, precision=jax.lax.Precision.HIGHEST)


def _mid(acc0, acc1, den0, den1, b, W, A):
    grid = NPAD // 512
    return pl.pallas_call(
        _tc_mid,
        grid=(grid,),
        in_specs=[
            pl.BlockSpec((512, D), lambda i: (i, 0)),
            pl.BlockSpec((512, D), lambda i: (i, 0)),
            pl.BlockSpec((512, 1), lambda i: (i, 0)),
            pl.BlockSpec((512, 1), lambda i: (i, 0)),
            pl.BlockSpec((1, D), lambda i: (0, 0)),
            pl.BlockSpec((D, D), lambda i: (0, 0)),
            pl.BlockSpec((D, 2), lambda i: (0, 0)),
        ],
        out_specs=[
            pl.BlockSpec((512, D), lambda i: (i, 0)),
            pl.BlockSpec((512, 2), lambda i: (i, 0)),
        ],
        out_shape=[
            jax.ShapeDtypeStruct((NPAD, D), jnp.float32),
            jax.ShapeDtypeStruct((NPAD, 2), jnp.float32),
        ],
    )(acc0, acc1, den0, den1, b, W, A)


def _tc_final(a0, a1, d0, d1, bb, out):
    den = d0[...] + d1[...] + 1e-16
    z = (a0[...] + a1[...]) * _recip(den) + bb[...]
    out[...] = jnp.maximum(z, 0.01 * z)


def _final(acc0, acc1, den0, den1, b):
    grid = NPAD // 512
    return pl.pallas_call(
        _tc_final,
        grid=(grid,),
        in_specs=[
            pl.BlockSpec((512, D), lambda i: (i, 0)),
            pl.BlockSpec((512, D), lambda i: (i, 0)),
            pl.BlockSpec((512, 1), lambda i: (i, 0)),
            pl.BlockSpec((512, 1), lambda i: (i, 0)),
            pl.BlockSpec((1, D), lambda i: (0, 0)),
        ],
        out_specs=pl.BlockSpec((512, D), lambda i: (i, 0)),
        out_shape=jax.ShapeDtypeStruct((NPAD, D), jnp.float32),
    )(acc0, acc1, den0, den1, b)


_EXP2_COEF = (1.3029887479e-06, 1.5463520236e-05, 1.5408674586e-04,
              1.3332134850e-03, 9.6180966089e-03, 5.5504143294e-02,
              2.4022651303e-01, 6.9314717838e-01, 9.9999999983e-01)
_LOG2E = 1.4426950408889634


def _soft_exp(x):
    t = jnp.maximum(x * _LOG2E, -120.0)
    n = t.astype(jnp.int32)
    f = t - n.astype(jnp.float32)
    p = jnp.full_like(x, _EXP2_COEF[0])
    for c in _EXP2_COEF[1:]:
        p = p * f + c
    scale = lax.bitcast_convert_type((n + 127) << 23, jnp.float32)
    return p * scale

def _sc_edge_body(srcm, dstm, asv, adv, mvec, h,
                  acc_out, den_out,
                  as_vm, ad_vm, m_vm, sidx, didx, pbuf,
                  hrow, dzbuf, acc_sh, den_sh):
    cid = lax.axis_index("c")
    sid = lax.axis_index("s")
    wid = cid * 16 + sid

    pltpu.sync_copy(asv, as_vm)
    pltpu.sync_copy(adv, ad_vm)
    pltpu.sync_copy(mvec, m_vm)

    zero16 = jnp.zeros((16,), jnp.float32)
    for j in range(CHUNK):
        for q in range(D // 16):
            hrow[j, pl.ds(q * 16, 16)] = zero16
    for j in range(ROWS_PER_TILE // 16):
        dzbuf[pl.ds(j * 16, 16)] = zero16
    rbase = sid * ROWS_PER_TILE
    for j in range(ROWS_PER_TILE // CHUNK):
        pltpu.sync_copy(hrow, acc_sh.at[pl.ds(rbase + j * CHUNK, CHUNK)])
    pltpu.sync_copy(dzbuf, den_sh.at[pl.ds(rbase, ROWS_PER_TILE)])
    plsc.subcore_barrier()

    mv = m_vm[...]

    def chunk_body(g, carry):
        row = wid * CPW + g
        pltpu.sync_copy(srcm.at[row], sidx.at[0])
        pltpu.sync_copy(dstm.at[row], didx.at[0])
        pltpu.sync_copy(h.at[sidx.at[0]], hrow)
        for q in range(CHUNK // 16):
            sv = sidx[0, pl.ds(q * 16, 16)]
            dv = didx[0, pl.ds(q * 16, 16)]
            as_v = plsc.load_gather(as_vm, [sv])
            ad_v = plsc.load_gather(ad_vm, [dv])
            s = as_v + ad_v
            e = jnp.maximum(s, 0.2 * s)
            t = mv + ad_v
            m = jnp.maximum(t, 0.2 * t)
            pbuf[pl.ds(q * 16, 16)] = _soft_exp(e - m)
        for r in range(CHUNK):
            pr = plsc.load_gather(pbuf, [jnp.full((16,), r, jnp.int32)])
            for q in range(D // 16):
                hrow[r, pl.ds(q * 16, 16)] = hrow[r, pl.ds(q * 16, 16)] * pr
        pltpu.sync_copy(hrow, acc_sh.at[didx.at[0]], add=True)
        pltpu.sync_copy(pbuf, den_sh.at[didx.at[0]], add=True)
        return carry

    lax.fori_loop(0, CPW, chunk_body, 0)
    plsc.subcore_barrier()

    for j in range(ROWS_PER_TILE // CHUNK):
        r0 = rbase + j * CHUNK
        pltpu.sync_copy(acc_sh.at[pl.ds(r0, CHUNK)],
                        acc_out.at[cid, pl.ds(r0, CHUNK)])
    pltpu.sync_copy(den_sh.at[pl.ds(rbase, ROWS_PER_TILE)],
                    den_out.at[cid, pl.ds(rbase, ROWS_PER_TILE)])


@functools.partial(jax.jit, static_argnums=())
def _sc_edge(srcm, dstm, asv, adv, mvec, h):
    mesh = plsc.VectorSubcoreMesh(core_axis_name="c", subcore_axis_name="s")
    f = pl.kernel(
        _sc_edge_body,
        out_type=[
            jax.ShapeDtypeStruct((2, NPAD, D), jnp.float32),
            jax.ShapeDtypeStruct((2, NPAD), jnp.float32),
        ],
        mesh=mesh,
        compiler_params=pltpu.CompilerParams(needs_layout_passes=False),
        scratch_types=[
            pltpu.VMEM((NPAD,), jnp.float32),
            pltpu.VMEM((NPAD,), jnp.float32),
            pltpu.VMEM((16,), jnp.float32),
            pltpu.VMEM((1, CHUNK), jnp.int32),
            pltpu.VMEM((1, CHUNK), jnp.int32),
            pltpu.VMEM((CHUNK,), jnp.float32),
            pltpu.VMEM((CHUNK, D), jnp.float32),
            pltpu.VMEM((ROWS_PER_TILE,), jnp.float32),
            pltpu.VMEM_SHARED((NPAD, D), jnp.float32),
            pltpu.VMEM_SHARED((NPAD,), jnp.float32),
        ],
    )
    return f(srcm, dstm, asv, adv, mvec, h)


def kernel(x, edge_index, W1, a_src1, a_dst1, b1, W2, a_src2, a_dst2, b2):
    f32 = jnp.float32
    xp = jnp.zeros((NPAD, D), f32).at[:N_NODES].set(x.astype(f32))

    ei = edge_index.astype(jnp.int32)
    loops = jnp.arange(N_NODES, dtype=jnp.int32)
    src = jnp.concatenate([ei[0], loops,
                           jnp.zeros((EPAD - E_TOT,), jnp.int32)])
    dst = jnp.concatenate([ei[1], loops,
                           jnp.full((EPAD - E_TOT,), N_NODES, jnp.int32)])
    srcm = src.reshape(EPAD // CHUNK, CHUNK)
    dstm = dst.reshape(EPAD // CHUNK, CHUNK)

    A1 = jnp.stack([a_src1.astype(f32), a_dst1.astype(f32)], axis=1)
    A2 = jnp.stack([a_src2.astype(f32), a_dst2.astype(f32)], axis=1)

    h1, av1 = _project(xp, W1.astype(f32), A1)
    as1 = av1[:, 0] + 0.0
    ad1 = av1[:, 1] + 0.0
    m1 = jnp.full((16,), jnp.max(as1), f32)
    acc1, den1 = _sc_edge(srcm, dstm, as1, ad1, m1, h1)

    h2, av2 = _mid(acc1[0], acc1[1], den1[0][:, None], den1[1][:, None],
                   b1.astype(f32)[None, :], W2.astype(f32), A2)
    as2 = av2[:, 0] + 0.0
    ad2 = av2[:, 1] + 0.0
    m2 = jnp.full((16,), jnp.max(as2), f32)
    acc2, den2 = _sc_edge(srcm, dstm, as2, ad2, m2, h2)

    out = _final(acc2[0], acc2[1], den2[0][:, None], den2[1][:, None],
                 b2.astype(f32)[None, :])
    return out[:N_NODES]
